# baseline (speedup 1.0000x reference)
import functools

import jax
import jax.numpy as jnp
from jax.experimental import pallas as pl
from jax.experimental.pallas import tpu as pltpu


def _cdiv(a, b):
    return -(-a // b)


def _mlp_t2_kernel(xa_ref, xb_ref, w1_ref, b1r_ref, w2t_ref, b2_ref, o_ref,
                   *, tbn):
    w1 = w1_ref[...]
    b1c = jnp.transpose(b1r_ref[...], (1, 0))
    w2t = w2t_ref[...]
    b2 = b2_ref[...]
    for k, x_ref in enumerate((xa_ref, xb_ref)):
        h = jax.lax.dot_general(
            w1, x_ref[...],
            dimension_numbers=(((0,), (0,)), ((), ())),
            preferred_element_type=jnp.float32)
        h = h + b1c
        h = jnp.maximum(h, 0.01 * h)
        y = jnp.dot(w2t, h, preferred_element_type=jnp.float32)
        o_ref[:, k * tbn:(k + 1) * tbn] = (y + b2).astype(o_ref.dtype)


@functools.partial(jax.jit, static_argnames=("block_n",))
def _run(x, w1, b1, w2, b2, *, block_n=32768):
    B, F = x.shape
    H = w1.shape[1]
    out_dtype = x.dtype

    xt = x.T
    w1c = w1.astype(jnp.float32)
    b1r = b1.reshape(1, H).astype(jnp.float32)
    w2t = w2.reshape(1, H).astype(jnp.float32)
    b2r = b2.reshape(1, 1).astype(jnp.float32)

    tbn = min(block_n, max(128, B // 2))
    tbn = max(128, (tbn // 128) * 128)
    grid = (_cdiv(B, 2 * tbn),)

    out = pl.pallas_call(
        functools.partial(_mlp_t2_kernel, tbn=tbn),
        out_shape=jax.ShapeDtypeStruct((1, B), out_dtype),
        grid=grid,
        in_specs=[
            pl.BlockSpec((F, tbn), lambda i: (0, 2 * i)),
            pl.BlockSpec((F, tbn), lambda i: (0, 2 * i + 1)),
            pl.BlockSpec((F, H), lambda i: (0, 0)),
            pl.BlockSpec((1, H), lambda i: (0, 0)),
            pl.BlockSpec((1, H), lambda i: (0, 0)),
            pl.BlockSpec((1, 1), lambda i: (0, 0)),
        ],
        out_specs=pl.BlockSpec((1, 2 * tbn), lambda i: (0, i)),
        compiler_params=pltpu.CompilerParams(
            dimension_semantics=("parallel",)),
    )(xt, xt, w1c, b1r, w2t, b2r)
    return out.reshape(B, 1)


def kernel(x, w1, b1, w2, b2):
    return _run(x, w1, b1, w2, b2)


# speedup vs baseline: 1.0118x; 1.0118x over previous
import functools

import jax
import jax.numpy as jnp
from jax.experimental import pallas as pl
from jax.experimental.pallas import tpu as pltpu


def _cdiv(a, b):
    return -(-a // b)


def _mlp_t_kernel(xt_ref, w1_ref, b1r_ref, w2t_ref, b2_ref, o_ref):
    h = jax.lax.dot_general(
        w1_ref[...], xt_ref[...],
        dimension_numbers=(((0,), (0,)), ((), ())),
        preferred_element_type=jnp.float32)
    h = h + jnp.transpose(b1r_ref[...], (1, 0))
    h = jnp.maximum(h, 0.01 * h)
    y = jnp.dot(w2t_ref[...], h,
                preferred_element_type=jnp.float32)
    o_ref[...] = (y + b2_ref[...]).astype(o_ref.dtype)


@functools.partial(jax.jit, static_argnames=("block_n",))
def _run(x, w1, b1, w2, b2, *, block_n=65536):
    B, F = x.shape
    H = w1.shape[1]
    out_dtype = x.dtype

    xt = x.T
    w1c = w1.astype(jnp.float32)
    b1r = b1.reshape(1, H).astype(jnp.float32)
    w2t = w2.reshape(1, H).astype(jnp.float32)
    b2r = b2.reshape(1, 1).astype(jnp.float32)

    tbn = min(block_n, B)
    if B >= 128:
        tbn = max(128, (tbn // 128) * 128)
    grid = (_cdiv(B, tbn),)

    out = pl.pallas_call(
        _mlp_t_kernel,
        out_shape=jax.ShapeDtypeStruct((1, B), out_dtype),
        grid=grid,
        in_specs=[
            pl.BlockSpec((F, tbn), lambda i: (0, i)),
            pl.BlockSpec((F, H), lambda i: (0, 0)),
            pl.BlockSpec((1, H), lambda i: (0, 0)),
            pl.BlockSpec((1, H), lambda i: (0, 0)),
            pl.BlockSpec((1, 1), lambda i: (0, 0)),
        ],
        out_specs=pl.BlockSpec((1, tbn), lambda i: (0, i)),
        compiler_params=pltpu.CompilerParams(
            dimension_semantics=("parallel",)),
    )(xt, w1c, b1r, w2t, b2r)
    return out.reshape(B, 1)


def kernel(x, w1, b1, w2, b2):
    return _run(x, w1, b1, w2, b2)
